# baseline (speedup 1.0000x reference)
import functools

import jax
import jax.numpy as jnp
from jax import lax
from jax.experimental import pallas as pl
from jax.experimental.pallas import tpu as pltpu
from jax.experimental.pallas import tpu_sc as plsc

_NC = 2
_NS = 16
_NW = _NC * _NS
_BLK = 256
_NB_PAD = 128


def _router_body(x_ref, w1_ref, b1_ref, w2_ref, b2_ref,
                 probs_ref, dest_ref, wnb_ref, eid_ref):
    x = x_ref[...]
    h = jnp.tanh(
        lax.dot_general(x, w1_ref[...], (((1,), (0,)), ((), ())),
                        preferred_element_type=jnp.float32) + b1_ref[...])
    logits = (
        lax.dot_general(h, w2_ref[...], (((1,), (0,)), ((), ())),
                        preferred_element_type=jnp.float32) + b2_ref[...])
    m = jnp.max(logits, axis=-1, keepdims=True)
    ex = jnp.exp(logits - m)
    probs = ex / jnp.sum(ex, axis=-1, keepdims=True)
    probs_ref[...] = probs

    t, e = probs.shape
    col = lax.broadcasted_iota(jnp.int32, (t, e), 1)
    m1 = jnp.max(probs, axis=-1, keepdims=True)
    i1 = jnp.argmax(probs, axis=-1)[:, None]
    probs_m = jnp.where(col == i1, -jnp.inf, probs)
    m2 = jnp.max(probs_m, axis=-1, keepdims=True)
    i2 = jnp.argmax(probs_m, axis=-1)[:, None]
    s = m1 + m2 + 1e-8
    w1n = m1 / s
    w2n = m2 / s

    mask1 = col == i1
    mask2 = col == i2
    mf = mask1.astype(jnp.float32) + mask2.astype(jnp.float32)
    cum = mf
    sh = 1
    while sh < t:
        shifted = jnp.concatenate(
            [jnp.zeros((sh, e), jnp.float32), lax.slice(cum, (0, 0), (t - sh, e))],
            axis=0)
        cum = cum + shifted
        sh *= 2
    cume = cum - mf
    counts = lax.slice(cum, (t - 1, 0), (t, e))
    padded = jnp.floor((counts + (_BLK - 1)) * (1.0 / _BLK)) * _BLK
    rt = lax.broadcasted_iota(jnp.int32, (e, e), 0)
    ct = lax.broadcasted_iota(jnp.int32, (e, e), 1)
    tril = (rt <= ct).astype(jnp.float32)
    pad_cum = lax.dot_general(padded, tril, (((1,), (0,)), ((), ())),
                              preferred_element_type=jnp.float32)
    pad_off = pad_cum - padded

    slot = cume + pad_off
    d1 = jnp.sum(jnp.where(mask1, slot, 0.0), axis=1).astype(jnp.int32)
    d2 = jnp.sum(jnp.where(mask2, slot, 0.0), axis=1).astype(jnp.int32)
    dest_ref[...] = jnp.concatenate(
        [d1.reshape(1, t), d2.reshape(1, t)], axis=0)

    wcat = jnp.concatenate([w1n, w2n], axis=0)
    wnb_ref[...] = wcat * jnp.ones((1, 128), jnp.float32)

    bi = lax.broadcasted_iota(jnp.int32, (_NB_PAD, e), 0).astype(jnp.float32)
    ge = (bi * _BLK >= pad_cum).astype(jnp.int32)
    eid = jnp.minimum(jnp.sum(ge, axis=1), e - 1)
    nbu = (lax.slice(pad_cum, (0, e - 1), (1, e)) * (1.0 / _BLK))
    nbu = nbu.astype(jnp.int32)[0, 0]
    pos = lax.broadcasted_iota(jnp.int32, (_NB_PAD,), 0)
    eid = jnp.where(pos == _NB_PAD - 1, nbu, eid)
    eid_ref[...] = eid.reshape(1, _NB_PAD)


def _moe_body(eid_ref, xs_ref, w1_ref, b1_ref, w2_ref, b2_ref, eo_ref):
    bi = pl.program_id(0)
    nbu = eid_ref[_NB_PAD - 1]

    @pl.when(bi < nbu)
    def _compute():
        xb = xs_ref[...].astype(jnp.bfloat16)
        dff = w1_ref.shape[2]
        fb = dff // 4
        acc = jnp.broadcast_to(b2_ref[0], eo_ref.shape).astype(jnp.float32)
        for fi in range(4):
            h = lax.dot_general(
                xb, w1_ref[0, :, fi * fb:(fi + 1) * fb],
                (((1,), (0,)), ((), ())),
                preferred_element_type=jnp.float32)
            hg = jax.nn.gelu(h + b1_ref[0, :, fi * fb:(fi + 1) * fb])
            acc = acc + lax.dot_general(
                hg.astype(jnp.bfloat16), w2_ref[0, fi * fb:(fi + 1) * fb, :],
                (((1,), (0,)), ((), ())),
                preferred_element_type=jnp.float32)
        eo_ref[...] = acc


def _make_dispatch(t, d, g, tpw, ch):
    mesh = plsc.VectorSubcoreMesh(core_axis_name="c", subcore_axis_name="s",
                                  num_cores=_NC, num_subcores=_NS)
    n = tpw // ch

    @functools.partial(
        pl.kernel, mesh=mesh,
        out_type=jax.ShapeDtypeStruct((g, d), jnp.float32),
        scratch_types=[[pltpu.VMEM((ch, d), jnp.float32)] * 2,
                       [pltpu.VMEM((ch,), jnp.int32)] * 2,
                       [pltpu.VMEM((ch,), jnp.int32)] * 2,
                       [pltpu.SemaphoreType.DMA] * 2,
                       [pltpu.SemaphoreType.DMA] * 2],
    )
    def dispatch(x_hbm, dest_hbm, xs_hbm, buf, idx1, idx2, gsem, ssem):
        wid = lax.axis_index("s") * _NC + lax.axis_index("c")
        tbase = wid * tpw

        def gather(ci):
            sb = ci % 2
            t0 = tbase + ci * ch
            pltpu.sync_copy(dest_hbm.at[pl.ds(t0, ch)], idx1[sb])
            pltpu.sync_copy(dest_hbm.at[pl.ds(t + t0, ch)], idx2[sb])
            return (pltpu.async_copy(x_hbm.at[pl.ds(t0, ch)], buf[sb],
                                     gsem[sb]),)

        def scatter(ci):
            sb = ci % 2
            return (pltpu.async_copy(buf[sb], xs_hbm.at[idx1[sb]], ssem[sb]),
                    pltpu.async_copy(buf[sb], xs_hbm.at[idx2[sb]], ssem[sb]))

        gps = {0: gather(0)}
        sps = {}
        for ci in range(n):
            if ci + 1 < n:
                if ci >= 1:
                    for cp in sps.pop(ci - 1):
                        cp.wait()
                gps[ci + 1] = gather(ci + 1)
            for cp in gps.pop(ci):
                cp.wait()
            sps[ci] = scatter(ci)
        for ci in sorted(sps):
            for cp in sps[ci]:
                cp.wait()

    return dispatch


def _make_combine(t, d, g, tpw, ch):
    mesh = plsc.VectorSubcoreMesh(core_axis_name="c", subcore_axis_name="s",
                                  num_cores=_NC, num_subcores=_NS)

    n = tpw // ch

    @functools.partial(
        pl.kernel, mesh=mesh,
        out_type=[jax.ShapeDtypeStruct((t, d), jnp.float32),
                  jax.ShapeDtypeStruct((t, d), jnp.float32)],
        scratch_types=[[pltpu.VMEM((ch, d), jnp.float32)] * 2,
                       [pltpu.VMEM((ch, d), jnp.float32)] * 2,
                       [pltpu.VMEM((ch,), jnp.int32)] * 2,
                       [pltpu.VMEM((ch,), jnp.int32)] * 2,
                       [pltpu.SemaphoreType.DMA] * 2,
                       [pltpu.SemaphoreType.DMA] * 2],
    )
    def combine(eo_hbm, dest_hbm, o1_hbm, o2_hbm, buf1, buf2, idx1, idx2,
                gsem, ssem):
        wid = lax.axis_index("s") * _NC + lax.axis_index("c")
        tbase = wid * tpw

        def gather(ci):
            sb = ci % 2
            t0 = tbase + ci * ch
            pltpu.sync_copy(dest_hbm.at[pl.ds(t0, ch)], idx1[sb])
            pltpu.sync_copy(dest_hbm.at[pl.ds(t + t0, ch)], idx2[sb])
            return (pltpu.async_copy(eo_hbm.at[idx1[sb]], buf1[sb], gsem[sb]),
                    pltpu.async_copy(eo_hbm.at[idx2[sb]], buf2[sb], gsem[sb]))

        def writeback(ci):
            sb = ci % 2
            t0 = tbase + ci * ch
            return (pltpu.async_copy(buf1[sb], o1_hbm.at[pl.ds(t0, ch)],
                                     ssem[sb]),
                    pltpu.async_copy(buf2[sb], o2_hbm.at[pl.ds(t0, ch)],
                                     ssem[sb]))

        gps = {0: gather(0)}
        sps = {}
        for ci in range(n):
            if ci + 1 < n:
                if ci >= 1:
                    for cp in sps.pop(ci - 1):
                        cp.wait()
                gps[ci + 1] = gather(ci + 1)
            for cp in gps.pop(ci):
                cp.wait()
            sps[ci] = writeback(ci)
        for ci in sorted(sps):
            for cp in sps[ci]:
                cp.wait()

    return combine


def _add_body(a_ref, b_ref, wa_ref, wb_ref, o_ref):
    o_ref[...] = (wa_ref[:, 0:1] * a_ref[...]
                  + wb_ref[:, 0:1] * b_ref[...])


def kernel(hidden_states, router_w1, router_b1, router_w2, router_b2,
           expert_w1, expert_b1, expert_w2, expert_b2):
    b, s, d = hidden_states.shape
    t = b * s
    e = router_w2.shape[1]
    dff = expert_w1.shape[2]
    rh = router_w1.shape[1]
    k = 2
    a = t * k
    g = a + e * _BLK
    nb = g // _BLK

    x = hidden_states.reshape(t, d)

    probs, dest, wnb, eid = pl.pallas_call(
        _router_body,
        grid=(1,),
        in_specs=[
            pl.BlockSpec((t, d), lambda i: (0, 0)),
            pl.BlockSpec((d, rh), lambda i: (0, 0)),
            pl.BlockSpec((1, rh), lambda i: (0, 0)),
            pl.BlockSpec((rh, e), lambda i: (0, 0)),
            pl.BlockSpec((1, e), lambda i: (0, 0)),
        ],
        out_specs=[
            pl.BlockSpec((t, e), lambda i: (0, 0)),
            pl.BlockSpec((k, t), lambda i: (0, 0)),
            pl.BlockSpec((a, 128), lambda i: (0, 0)),
            pl.BlockSpec((1, _NB_PAD), lambda i: (0, 0)),
        ],
        out_shape=[
            jax.ShapeDtypeStruct((t, e), jnp.float32),
            jax.ShapeDtypeStruct((k, t), jnp.int32),
            jax.ShapeDtypeStruct((a, 128), jnp.float32),
            jax.ShapeDtypeStruct((1, _NB_PAD), jnp.int32),
        ],
    )(x, router_w1, router_b1.reshape(1, rh), router_w2,
      router_b2.reshape(1, e))

    dest_flat = dest.reshape(a)
    eid_flat = eid.reshape(_NB_PAD)

    tpw = t // _NW
    xs = _make_dispatch(t, d, g, tpw, min(32, tpw))(x, dest_flat)

    w1b = expert_w1.astype(jnp.bfloat16)
    w2b = expert_w2.astype(jnp.bfloat16)

    grid_spec = pltpu.PrefetchScalarGridSpec(
        num_scalar_prefetch=1,
        grid=(nb,),
        in_specs=[
            pl.BlockSpec((_BLK, d), lambda bi, eid_r: (bi, 0)),
            pl.BlockSpec((1, d, dff), lambda bi, eid_r: (eid_r[bi], 0, 0)),
            pl.BlockSpec((1, 1, dff), lambda bi, eid_r: (eid_r[bi], 0, 0)),
            pl.BlockSpec((1, dff, d), lambda bi, eid_r: (eid_r[bi], 0, 0)),
            pl.BlockSpec((1, 1, d), lambda bi, eid_r: (eid_r[bi], 0, 0)),
        ],
        out_specs=pl.BlockSpec((_BLK, d), lambda bi, eid_r: (bi, 0)),
    )
    eo = pl.pallas_call(
        _moe_body,
        grid_spec=grid_spec,
        out_shape=jax.ShapeDtypeStruct((g, d), jnp.float32),
    )(eid_flat, xs, w1b, expert_b1.reshape(e, 1, dff), w2b,
      expert_b2.reshape(e, 1, d))

    eo1, eo2 = _make_combine(t, d, g, tpw, min(16, tpw))(eo, dest_flat)

    tb = min(1024, t)
    nblk_w = t // tb
    out = pl.pallas_call(
        _add_body,
        grid=(t // tb,),
        in_specs=[
            pl.BlockSpec((tb, d), lambda i: (i, 0)),
            pl.BlockSpec((tb, d), lambda i: (i, 0)),
            pl.BlockSpec((tb, 128), lambda i: (i, 0)),
            pl.BlockSpec((tb, 128), lambda i, _n=nblk_w: (i + _n, 0)),
        ],
        out_specs=pl.BlockSpec((tb, d), lambda i: (i, 0)),
        out_shape=jax.ShapeDtypeStruct((t, d), jnp.float32),
    )(eo1, eo2, wnb, wnb)

    return out.reshape(b, s, d), probs.reshape(b, s, e)


# speedup vs baseline: 1.0736x; 1.0736x over previous
import functools

import jax
import jax.numpy as jnp
from jax import lax
from jax.experimental import pallas as pl
from jax.experimental.pallas import tpu as pltpu
from jax.experimental.pallas import tpu_sc as plsc

_NC = 2
_NS = 16
_NW = _NC * _NS
_BLK = 256
_NB_PAD = 128


def _router_body(x_ref, w1_ref, b1_ref, w2_ref, b2_ref,
                 probs_ref, dest_ref, wnb_ref, eid_ref):
    x = x_ref[...]
    h = jnp.tanh(
        lax.dot_general(x, w1_ref[...], (((1,), (0,)), ((), ())),
                        preferred_element_type=jnp.float32) + b1_ref[...])
    logits = (
        lax.dot_general(h, w2_ref[...], (((1,), (0,)), ((), ())),
                        preferred_element_type=jnp.float32) + b2_ref[...])
    m = jnp.max(logits, axis=-1, keepdims=True)
    ex = jnp.exp(logits - m)
    probs = ex / jnp.sum(ex, axis=-1, keepdims=True)
    probs_ref[...] = probs

    t, e = probs.shape
    col = lax.broadcasted_iota(jnp.int32, (t, e), 1)
    m1 = jnp.max(probs, axis=-1, keepdims=True)
    i1 = jnp.argmax(probs, axis=-1)[:, None]
    probs_m = jnp.where(col == i1, -jnp.inf, probs)
    m2 = jnp.max(probs_m, axis=-1, keepdims=True)
    i2 = jnp.argmax(probs_m, axis=-1)[:, None]
    s = m1 + m2 + 1e-8
    w1n = m1 / s
    w2n = m2 / s

    mask1 = col == i1
    mask2 = col == i2
    mf = mask1.astype(jnp.float32) + mask2.astype(jnp.float32)
    cum = mf
    sh = 1
    while sh < t:
        shifted = jnp.concatenate(
            [jnp.zeros((sh, e), jnp.float32), lax.slice(cum, (0, 0), (t - sh, e))],
            axis=0)
        cum = cum + shifted
        sh *= 2
    cume = cum - mf
    counts = lax.slice(cum, (t - 1, 0), (t, e))
    padded = jnp.floor((counts + (_BLK - 1)) * (1.0 / _BLK)) * _BLK
    rt = lax.broadcasted_iota(jnp.int32, (e, e), 0)
    ct = lax.broadcasted_iota(jnp.int32, (e, e), 1)
    tril = (rt <= ct).astype(jnp.float32)
    pad_cum = lax.dot_general(padded, tril, (((1,), (0,)), ((), ())),
                              preferred_element_type=jnp.float32)
    pad_off = pad_cum - padded

    slot = cume + pad_off
    d1 = jnp.sum(jnp.where(mask1, slot, 0.0), axis=1).astype(jnp.int32)
    d2 = jnp.sum(jnp.where(mask2, slot, 0.0), axis=1).astype(jnp.int32)
    dest_ref[...] = jnp.concatenate(
        [d1.reshape(1, t), d2.reshape(1, t)], axis=0)

    wcat = jnp.concatenate([w1n, w2n], axis=0)
    wnb_ref[...] = wcat * jnp.ones((1, 128), jnp.float32)

    bi = lax.broadcasted_iota(jnp.int32, (_NB_PAD, e), 0).astype(jnp.float32)
    ge = (bi * _BLK >= pad_cum).astype(jnp.int32)
    eid = jnp.minimum(jnp.sum(ge, axis=1), e - 1)
    nbu = (lax.slice(pad_cum, (0, e - 1), (1, e)) * (1.0 / _BLK))
    nbu = nbu.astype(jnp.int32)[0, 0]
    pos = lax.broadcasted_iota(jnp.int32, (_NB_PAD,), 0)
    eid = jnp.where(pos == _NB_PAD - 1, nbu, eid)
    eid_ref[...] = eid.reshape(1, _NB_PAD)


def _moe_body(eid_ref, xs_ref, w1_ref, b1_ref, w2_ref, b2_ref, eo_ref,
              w1c_ref, w2c_ref, le_ref):
    bi = pl.program_id(0)
    nbu = eid_ref[_NB_PAD - 1]

    @pl.when(bi < nbu)
    def _compute():
        eid = eid_ref[bi]
        cvt = jnp.logical_or(bi == 0, eid != le_ref[0])
        xb = xs_ref[...].astype(jnp.bfloat16)
        dff = w1_ref.shape[2]
        fb = dff // 4
        acc = jnp.broadcast_to(b2_ref[0], eo_ref.shape).astype(jnp.float32)
        for fi in range(4):
            fs = pl.ds(fi * fb, fb)

            @pl.when(cvt)
            def _cvt():
                w1c_ref[:, fs] = w1_ref[0, :, fs].astype(jnp.bfloat16)
                w2c_ref[fs, :] = w2_ref[0, fs, :].astype(jnp.bfloat16)

            h = lax.dot_general(
                xb, w1c_ref[:, fs],
                (((1,), (0,)), ((), ())),
                preferred_element_type=jnp.float32)
            hg = jax.nn.gelu(h + b1_ref[0, :, fs])
            acc = acc + lax.dot_general(
                hg.astype(jnp.bfloat16), w2c_ref[fs, :],
                (((1,), (0,)), ((), ())),
                preferred_element_type=jnp.float32)
        eo_ref[...] = acc
        le_ref[0] = eid


def _make_dispatch(t, d, g, tpw, ch):
    mesh = plsc.VectorSubcoreMesh(core_axis_name="c", subcore_axis_name="s",
                                  num_cores=_NC, num_subcores=_NS)
    n = tpw // ch

    @functools.partial(
        pl.kernel, mesh=mesh,
        out_type=jax.ShapeDtypeStruct((g, d), jnp.float32),
        scratch_types=[[pltpu.VMEM((ch, d), jnp.float32)] * 2,
                       [pltpu.VMEM((ch,), jnp.int32)] * 2,
                       [pltpu.VMEM((ch,), jnp.int32)] * 2,
                       [pltpu.SemaphoreType.DMA] * 2,
                       [pltpu.SemaphoreType.DMA] * 2],
    )
    def dispatch(x_hbm, dest_hbm, xs_hbm, buf, idx1, idx2, gsem, ssem):
        wid = lax.axis_index("s") * _NC + lax.axis_index("c")
        tbase = wid * tpw

        def gather(ci):
            sb = ci % 2
            t0 = tbase + ci * ch
            pltpu.sync_copy(dest_hbm.at[pl.ds(t0, ch)], idx1[sb])
            pltpu.sync_copy(dest_hbm.at[pl.ds(t + t0, ch)], idx2[sb])
            return (pltpu.async_copy(x_hbm.at[pl.ds(t0, ch)], buf[sb],
                                     gsem[sb]),)

        def scatter(ci):
            sb = ci % 2
            return (pltpu.async_copy(buf[sb], xs_hbm.at[idx1[sb]], ssem[sb]),
                    pltpu.async_copy(buf[sb], xs_hbm.at[idx2[sb]], ssem[sb]))

        gps = {0: gather(0)}
        sps = {}
        for ci in range(n):
            if ci + 1 < n:
                if ci >= 1:
                    for cp in sps.pop(ci - 1):
                        cp.wait()
                gps[ci + 1] = gather(ci + 1)
            for cp in gps.pop(ci):
                cp.wait()
            sps[ci] = scatter(ci)
        for ci in sorted(sps):
            for cp in sps[ci]:
                cp.wait()

    return dispatch


def _make_combine(t, d, g, tpw, ch):
    mesh = plsc.VectorSubcoreMesh(core_axis_name="c", subcore_axis_name="s",
                                  num_cores=_NC, num_subcores=_NS)

    n = tpw // ch

    @functools.partial(
        pl.kernel, mesh=mesh,
        out_type=[jax.ShapeDtypeStruct((t, d), jnp.float32),
                  jax.ShapeDtypeStruct((t, d), jnp.float32)],
        scratch_types=[[pltpu.VMEM((ch, d), jnp.float32)] * 2,
                       [pltpu.VMEM((ch, d), jnp.float32)] * 2,
                       [pltpu.VMEM((ch,), jnp.int32)] * 2,
                       [pltpu.VMEM((ch,), jnp.int32)] * 2,
                       [pltpu.SemaphoreType.DMA] * 2,
                       [pltpu.SemaphoreType.DMA] * 2],
    )
    def combine(eo_hbm, dest_hbm, o1_hbm, o2_hbm, buf1, buf2, idx1, idx2,
                gsem, ssem):
        wid = lax.axis_index("s") * _NC + lax.axis_index("c")
        tbase = wid * tpw

        def gather(ci):
            sb = ci % 2
            t0 = tbase + ci * ch
            pltpu.sync_copy(dest_hbm.at[pl.ds(t0, ch)], idx1[sb])
            pltpu.sync_copy(dest_hbm.at[pl.ds(t + t0, ch)], idx2[sb])
            return (pltpu.async_copy(eo_hbm.at[idx1[sb]], buf1[sb], gsem[sb]),
                    pltpu.async_copy(eo_hbm.at[idx2[sb]], buf2[sb], gsem[sb]))

        def writeback(ci):
            sb = ci % 2
            t0 = tbase + ci * ch
            return (pltpu.async_copy(buf1[sb], o1_hbm.at[pl.ds(t0, ch)],
                                     ssem[sb]),
                    pltpu.async_copy(buf2[sb], o2_hbm.at[pl.ds(t0, ch)],
                                     ssem[sb]))

        gps = {0: gather(0)}
        sps = {}
        for ci in range(n):
            if ci + 1 < n:
                if ci >= 1:
                    for cp in sps.pop(ci - 1):
                        cp.wait()
                gps[ci + 1] = gather(ci + 1)
            for cp in gps.pop(ci):
                cp.wait()
            sps[ci] = writeback(ci)
        for ci in sorted(sps):
            for cp in sps[ci]:
                cp.wait()

    return combine


def _add_body(a_ref, b_ref, wa_ref, wb_ref, o_ref):
    o_ref[...] = (wa_ref[:, 0:1] * a_ref[...]
                  + wb_ref[:, 0:1] * b_ref[...])


def kernel(hidden_states, router_w1, router_b1, router_w2, router_b2,
           expert_w1, expert_b1, expert_w2, expert_b2):
    b, s, d = hidden_states.shape
    t = b * s
    e = router_w2.shape[1]
    dff = expert_w1.shape[2]
    rh = router_w1.shape[1]
    k = 2
    a = t * k
    g = a + e * _BLK
    nb = g // _BLK

    x = hidden_states.reshape(t, d)

    probs, dest, wnb, eid = pl.pallas_call(
        _router_body,
        grid=(1,),
        in_specs=[
            pl.BlockSpec((t, d), lambda i: (0, 0)),
            pl.BlockSpec((d, rh), lambda i: (0, 0)),
            pl.BlockSpec((1, rh), lambda i: (0, 0)),
            pl.BlockSpec((rh, e), lambda i: (0, 0)),
            pl.BlockSpec((1, e), lambda i: (0, 0)),
        ],
        out_specs=[
            pl.BlockSpec((t, e), lambda i: (0, 0)),
            pl.BlockSpec((k, t), lambda i: (0, 0)),
            pl.BlockSpec((a, 128), lambda i: (0, 0)),
            pl.BlockSpec((1, _NB_PAD), lambda i: (0, 0)),
        ],
        out_shape=[
            jax.ShapeDtypeStruct((t, e), jnp.float32),
            jax.ShapeDtypeStruct((k, t), jnp.int32),
            jax.ShapeDtypeStruct((a, 128), jnp.float32),
            jax.ShapeDtypeStruct((1, _NB_PAD), jnp.int32),
        ],
    )(x, router_w1, router_b1.reshape(1, rh), router_w2,
      router_b2.reshape(1, e))

    dest_flat = dest.reshape(a)
    eid_flat = eid.reshape(_NB_PAD)

    tpw = t // _NW
    xs = _make_dispatch(t, d, g, tpw, min(32, tpw))(x, dest_flat)

    grid_spec = pltpu.PrefetchScalarGridSpec(
        num_scalar_prefetch=1,
        grid=(nb,),
        scratch_shapes=[pltpu.VMEM((d, dff), jnp.bfloat16),
                        pltpu.VMEM((dff, d), jnp.bfloat16),
                        pltpu.SMEM((1,), jnp.int32)],
        in_specs=[
            pl.BlockSpec((_BLK, d), lambda bi, eid_r: (bi, 0)),
            pl.BlockSpec((1, d, dff), lambda bi, eid_r: (eid_r[bi], 0, 0)),
            pl.BlockSpec((1, 1, dff), lambda bi, eid_r: (eid_r[bi], 0, 0)),
            pl.BlockSpec((1, dff, d), lambda bi, eid_r: (eid_r[bi], 0, 0)),
            pl.BlockSpec((1, 1, d), lambda bi, eid_r: (eid_r[bi], 0, 0)),
        ],
        out_specs=pl.BlockSpec((_BLK, d), lambda bi, eid_r: (bi, 0)),
    )
    eo = pl.pallas_call(
        _moe_body,
        grid_spec=grid_spec,
        out_shape=jax.ShapeDtypeStruct((g, d), jnp.float32),
    )(eid_flat, xs, expert_w1, expert_b1.reshape(e, 1, dff), expert_w2,
      expert_b2.reshape(e, 1, d))

    eo1, eo2 = _make_combine(t, d, g, tpw, min(16, tpw))(eo, dest_flat)

    tb = min(1024, t)
    nblk_w = t // tb
    out = pl.pallas_call(
        _add_body,
        grid=(t // tb,),
        in_specs=[
            pl.BlockSpec((tb, d), lambda i: (i, 0)),
            pl.BlockSpec((tb, d), lambda i: (i, 0)),
            pl.BlockSpec((tb, 128), lambda i: (i, 0)),
            pl.BlockSpec((tb, 128), lambda i, _n=nblk_w: (i + _n, 0)),
        ],
        out_specs=pl.BlockSpec((tb, d), lambda i: (i, 0)),
        out_shape=jax.ShapeDtypeStruct((t, d), jnp.float32),
    )(eo1, eo2, wnb, wnb)

    return out.reshape(b, s, d), probs.reshape(b, s, e)


# speedup vs baseline: 1.1709x; 1.0906x over previous
import functools

import jax
import jax.numpy as jnp
from jax import lax
from jax.experimental import pallas as pl
from jax.experimental.pallas import tpu as pltpu
from jax.experimental.pallas import tpu_sc as plsc

_NC = 2
_NS = 16
_NW = _NC * _NS
_BLK = 512
_NB_PAD = 128


def _router_body(x_ref, w1_ref, b1_ref, w2_ref, b2_ref,
                 probs_ref, dest_ref, wnb_ref, eid_ref):
    x = x_ref[...]
    h = jnp.tanh(
        lax.dot_general(x, w1_ref[...], (((1,), (0,)), ((), ())),
                        preferred_element_type=jnp.float32) + b1_ref[...])
    logits = (
        lax.dot_general(h, w2_ref[...], (((1,), (0,)), ((), ())),
                        preferred_element_type=jnp.float32) + b2_ref[...])
    m = jnp.max(logits, axis=-1, keepdims=True)
    ex = jnp.exp(logits - m)
    probs = ex / jnp.sum(ex, axis=-1, keepdims=True)
    probs_ref[...] = probs

    t, e = probs.shape
    col = lax.broadcasted_iota(jnp.int32, (t, e), 1)
    m1 = jnp.max(probs, axis=-1, keepdims=True)
    i1 = jnp.argmax(probs, axis=-1)[:, None]
    probs_m = jnp.where(col == i1, -jnp.inf, probs)
    m2 = jnp.max(probs_m, axis=-1, keepdims=True)
    i2 = jnp.argmax(probs_m, axis=-1)[:, None]
    s = m1 + m2 + 1e-8
    w1n = m1 / s
    w2n = m2 / s

    mask1 = col == i1
    mask2 = col == i2
    mf = mask1.astype(jnp.float32) + mask2.astype(jnp.float32)
    cum = mf
    sh = 1
    while sh < t:
        shifted = jnp.concatenate(
            [jnp.zeros((sh, e), jnp.float32), lax.slice(cum, (0, 0), (t - sh, e))],
            axis=0)
        cum = cum + shifted
        sh *= 2
    cume = cum - mf
    counts = lax.slice(cum, (t - 1, 0), (t, e))
    padded = jnp.floor((counts + (_BLK - 1)) * (1.0 / _BLK)) * _BLK
    rt = lax.broadcasted_iota(jnp.int32, (e, e), 0)
    ct = lax.broadcasted_iota(jnp.int32, (e, e), 1)
    tril = (rt <= ct).astype(jnp.float32)
    pad_cum = lax.dot_general(padded, tril, (((1,), (0,)), ((), ())),
                              preferred_element_type=jnp.float32)
    pad_off = pad_cum - padded

    slot = cume + pad_off
    d1 = jnp.sum(jnp.where(mask1, slot, 0.0), axis=1).astype(jnp.int32)
    d2 = jnp.sum(jnp.where(mask2, slot, 0.0), axis=1).astype(jnp.int32)
    dest_ref[...] = jnp.concatenate(
        [d1.reshape(1, t), d2.reshape(1, t)], axis=0)

    wcat = jnp.concatenate([w1n, w2n], axis=0)
    wnb_ref[...] = wcat * jnp.ones((1, 128), jnp.float32)

    bi = lax.broadcasted_iota(jnp.int32, (_NB_PAD, e), 0).astype(jnp.float32)
    ge = (bi * _BLK >= pad_cum).astype(jnp.int32)
    eid = jnp.minimum(jnp.sum(ge, axis=1), e - 1)
    nbu = (lax.slice(pad_cum, (0, e - 1), (1, e)) * (1.0 / _BLK))
    nbu = nbu.astype(jnp.int32)[0, 0]
    pos = lax.broadcasted_iota(jnp.int32, (_NB_PAD,), 0)
    eid = jnp.where(pos == _NB_PAD - 1, nbu, eid)
    eid_ref[...] = eid.reshape(1, _NB_PAD)


def _moe_body(eid_ref, xs_ref, w1_ref, b1_ref, w2_ref, b2_ref, eo_ref,
              w1c_ref, w2c_ref, le_ref):
    bi = pl.program_id(0)
    nbu = eid_ref[_NB_PAD - 1]

    @pl.when(bi < nbu)
    def _compute():
        eid = eid_ref[bi]
        cvt = jnp.logical_or(bi == 0, eid != le_ref[0])
        xb = xs_ref[...].astype(jnp.bfloat16)
        dff = w1_ref.shape[2]
        fb = dff // 4
        acc = jnp.broadcast_to(b2_ref[0], eo_ref.shape).astype(jnp.float32)
        for fi in range(4):
            fs = pl.ds(fi * fb, fb)

            @pl.when(cvt)
            def _cvt():
                w1c_ref[:, fs] = w1_ref[0, :, fs].astype(jnp.bfloat16)
                w2c_ref[fs, :] = w2_ref[0, fs, :].astype(jnp.bfloat16)

            h = lax.dot_general(
                xb, w1c_ref[:, fs],
                (((1,), (0,)), ((), ())),
                preferred_element_type=jnp.float32)
            hg = jax.nn.gelu(h + b1_ref[0, :, fs])
            acc = acc + lax.dot_general(
                hg.astype(jnp.bfloat16), w2c_ref[fs, :],
                (((1,), (0,)), ((), ())),
                preferred_element_type=jnp.float32)
        eo_ref[...] = acc
        le_ref[0] = eid


def _make_dispatch(t, d, g, tpw, ch):
    mesh = plsc.VectorSubcoreMesh(core_axis_name="c", subcore_axis_name="s",
                                  num_cores=_NC, num_subcores=_NS)
    n = tpw // ch

    @functools.partial(
        pl.kernel, mesh=mesh,
        out_type=jax.ShapeDtypeStruct((g, d), jnp.float32),
        scratch_types=[[pltpu.VMEM((ch, d), jnp.float32)] * 2,
                       [pltpu.VMEM((ch,), jnp.int32)] * 2,
                       [pltpu.VMEM((ch,), jnp.int32)] * 2,
                       [pltpu.SemaphoreType.DMA] * 2,
                       [pltpu.SemaphoreType.DMA] * 2],
    )
    def dispatch(x_hbm, dest_hbm, xs_hbm, buf, idx1, idx2, gsem, ssem):
        wid = lax.axis_index("s") * _NC + lax.axis_index("c")
        tbase = wid * tpw

        def gather(ci):
            sb = ci % 2
            t0 = tbase + ci * ch
            pltpu.sync_copy(dest_hbm.at[pl.ds(t0, ch)], idx1[sb])
            pltpu.sync_copy(dest_hbm.at[pl.ds(t + t0, ch)], idx2[sb])
            return (pltpu.async_copy(x_hbm.at[pl.ds(t0, ch)], buf[sb],
                                     gsem[sb]),)

        def scatter(ci):
            sb = ci % 2
            return (pltpu.async_copy(buf[sb], xs_hbm.at[idx1[sb]], ssem[sb]),
                    pltpu.async_copy(buf[sb], xs_hbm.at[idx2[sb]], ssem[sb]))

        gps = {0: gather(0)}
        sps = {}
        for ci in range(n):
            if ci + 1 < n:
                if ci >= 1:
                    for cp in sps.pop(ci - 1):
                        cp.wait()
                gps[ci + 1] = gather(ci + 1)
            for cp in gps.pop(ci):
                cp.wait()
            sps[ci] = scatter(ci)
        for ci in sorted(sps):
            for cp in sps[ci]:
                cp.wait()

    return dispatch


def _make_combine(t, d, g, tpw, ch):
    mesh = plsc.VectorSubcoreMesh(core_axis_name="c", subcore_axis_name="s",
                                  num_cores=_NC, num_subcores=_NS)

    n = tpw // ch

    @functools.partial(
        pl.kernel, mesh=mesh,
        out_type=[jax.ShapeDtypeStruct((t, d), jnp.float32),
                  jax.ShapeDtypeStruct((t, d), jnp.float32)],
        scratch_types=[[pltpu.VMEM((ch, d), jnp.float32)] * 2,
                       [pltpu.VMEM((ch, d), jnp.float32)] * 2,
                       [pltpu.VMEM((ch,), jnp.int32)] * 2,
                       [pltpu.VMEM((ch,), jnp.int32)] * 2,
                       [pltpu.SemaphoreType.DMA] * 2,
                       [pltpu.SemaphoreType.DMA] * 2],
    )
    def combine(eo_hbm, dest_hbm, o1_hbm, o2_hbm, buf1, buf2, idx1, idx2,
                gsem, ssem):
        wid = lax.axis_index("s") * _NC + lax.axis_index("c")
        tbase = wid * tpw

        def gather(ci):
            sb = ci % 2
            t0 = tbase + ci * ch
            pltpu.sync_copy(dest_hbm.at[pl.ds(t0, ch)], idx1[sb])
            pltpu.sync_copy(dest_hbm.at[pl.ds(t + t0, ch)], idx2[sb])
            return (pltpu.async_copy(eo_hbm.at[idx1[sb]], buf1[sb], gsem[sb]),
                    pltpu.async_copy(eo_hbm.at[idx2[sb]], buf2[sb], gsem[sb]))

        def writeback(ci):
            sb = ci % 2
            t0 = tbase + ci * ch
            return (pltpu.async_copy(buf1[sb], o1_hbm.at[pl.ds(t0, ch)],
                                     ssem[sb]),
                    pltpu.async_copy(buf2[sb], o2_hbm.at[pl.ds(t0, ch)],
                                     ssem[sb]))

        gps = {0: gather(0)}
        sps = {}
        for ci in range(n):
            if ci + 1 < n:
                if ci >= 1:
                    for cp in sps.pop(ci - 1):
                        cp.wait()
                gps[ci + 1] = gather(ci + 1)
            for cp in gps.pop(ci):
                cp.wait()
            sps[ci] = writeback(ci)
        for ci in sorted(sps):
            for cp in sps[ci]:
                cp.wait()

    return combine


def _add_body(a_ref, b_ref, wa_ref, wb_ref, o_ref):
    o_ref[...] = (wa_ref[:, 0:1] * a_ref[...]
                  + wb_ref[:, 0:1] * b_ref[...])


def kernel(hidden_states, router_w1, router_b1, router_w2, router_b2,
           expert_w1, expert_b1, expert_w2, expert_b2):
    b, s, d = hidden_states.shape
    t = b * s
    e = router_w2.shape[1]
    dff = expert_w1.shape[2]
    rh = router_w1.shape[1]
    k = 2
    a = t * k
    g = a + e * _BLK
    nb = g // _BLK

    x = hidden_states.reshape(t, d)

    probs, dest, wnb, eid = pl.pallas_call(
        _router_body,
        grid=(1,),
        in_specs=[
            pl.BlockSpec((t, d), lambda i: (0, 0)),
            pl.BlockSpec((d, rh), lambda i: (0, 0)),
            pl.BlockSpec((1, rh), lambda i: (0, 0)),
            pl.BlockSpec((rh, e), lambda i: (0, 0)),
            pl.BlockSpec((1, e), lambda i: (0, 0)),
        ],
        out_specs=[
            pl.BlockSpec((t, e), lambda i: (0, 0)),
            pl.BlockSpec((k, t), lambda i: (0, 0)),
            pl.BlockSpec((a, 128), lambda i: (0, 0)),
            pl.BlockSpec((1, _NB_PAD), lambda i: (0, 0)),
        ],
        out_shape=[
            jax.ShapeDtypeStruct((t, e), jnp.float32),
            jax.ShapeDtypeStruct((k, t), jnp.int32),
            jax.ShapeDtypeStruct((a, 128), jnp.float32),
            jax.ShapeDtypeStruct((1, _NB_PAD), jnp.int32),
        ],
    )(x, router_w1, router_b1.reshape(1, rh), router_w2,
      router_b2.reshape(1, e))

    dest_flat = dest.reshape(a)
    eid_flat = eid.reshape(_NB_PAD)

    tpw = t // _NW
    xs = _make_dispatch(t, d, g, tpw, min(32, tpw))(x, dest_flat)

    grid_spec = pltpu.PrefetchScalarGridSpec(
        num_scalar_prefetch=1,
        grid=(nb,),
        scratch_shapes=[pltpu.VMEM((d, dff), jnp.bfloat16),
                        pltpu.VMEM((dff, d), jnp.bfloat16),
                        pltpu.SMEM((1,), jnp.int32)],
        in_specs=[
            pl.BlockSpec((_BLK, d), lambda bi, eid_r: (bi, 0)),
            pl.BlockSpec((1, d, dff), lambda bi, eid_r: (eid_r[bi], 0, 0)),
            pl.BlockSpec((1, 1, dff), lambda bi, eid_r: (eid_r[bi], 0, 0)),
            pl.BlockSpec((1, dff, d), lambda bi, eid_r: (eid_r[bi], 0, 0)),
            pl.BlockSpec((1, 1, d), lambda bi, eid_r: (eid_r[bi], 0, 0)),
        ],
        out_specs=pl.BlockSpec((_BLK, d), lambda bi, eid_r: (bi, 0)),
    )
    eo = pl.pallas_call(
        _moe_body,
        grid_spec=grid_spec,
        out_shape=jax.ShapeDtypeStruct((g, d), jnp.float32),
    )(eid_flat, xs, expert_w1, expert_b1.reshape(e, 1, dff), expert_w2,
      expert_b2.reshape(e, 1, d))

    eo1, eo2 = _make_combine(t, d, g, tpw, min(16, tpw))(eo, dest_flat)

    tb = min(1024, t)
    nblk_w = t // tb
    out = pl.pallas_call(
        _add_body,
        grid=(t // tb,),
        in_specs=[
            pl.BlockSpec((tb, d), lambda i: (i, 0)),
            pl.BlockSpec((tb, d), lambda i: (i, 0)),
            pl.BlockSpec((tb, 128), lambda i: (i, 0)),
            pl.BlockSpec((tb, 128), lambda i, _n=nblk_w: (i + _n, 0)),
        ],
        out_specs=pl.BlockSpec((tb, d), lambda i: (i, 0)),
        out_shape=jax.ShapeDtypeStruct((t, d), jnp.float32),
    )(eo1, eo2, wnb, wnb)

    return out.reshape(b, s, d), probs.reshape(b, s, e)


# speedup vs baseline: 1.2170x; 1.0393x over previous
import functools

import jax
import jax.numpy as jnp
from jax import lax
from jax.experimental import pallas as pl
from jax.experimental.pallas import tpu as pltpu
from jax.experimental.pallas import tpu_sc as plsc

_NC = 2
_NS = 16
_NW = _NC * _NS
_BLK = 512
_NB_PAD = 128


def _router_body(x_ref, w1_ref, b1_ref, w2_ref, b2_ref,
                 probs_ref, dest_ref, wnb_ref, eid_ref):
    x = x_ref[...]
    h = jnp.tanh(
        lax.dot_general(x, w1_ref[...], (((1,), (0,)), ((), ())),
                        preferred_element_type=jnp.float32) + b1_ref[...])
    logits = (
        lax.dot_general(h, w2_ref[...], (((1,), (0,)), ((), ())),
                        preferred_element_type=jnp.float32) + b2_ref[...])
    m = jnp.max(logits, axis=-1, keepdims=True)
    ex = jnp.exp(logits - m)
    probs = ex / jnp.sum(ex, axis=-1, keepdims=True)
    probs_ref[...] = probs

    t, e = probs.shape
    col = lax.broadcasted_iota(jnp.int32, (t, e), 1)
    m1 = jnp.max(probs, axis=-1, keepdims=True)
    i1 = jnp.argmax(probs, axis=-1)[:, None]
    probs_m = jnp.where(col == i1, -jnp.inf, probs)
    m2 = jnp.max(probs_m, axis=-1, keepdims=True)
    i2 = jnp.argmax(probs_m, axis=-1)[:, None]
    s = m1 + m2 + 1e-8
    w1n = m1 / s
    w2n = m2 / s

    mask1 = col == i1
    mask2 = col == i2
    mf = mask1.astype(jnp.float32) + mask2.astype(jnp.float32)
    cum = mf
    sh = 1
    while sh < t:
        shifted = jnp.concatenate(
            [jnp.zeros((sh, e), jnp.float32), lax.slice(cum, (0, 0), (t - sh, e))],
            axis=0)
        cum = cum + shifted
        sh *= 2
    cume = cum - mf
    counts = lax.slice(cum, (t - 1, 0), (t, e))
    padded = jnp.floor((counts + (_BLK - 1)) * (1.0 / _BLK)) * _BLK
    rt = lax.broadcasted_iota(jnp.int32, (e, e), 0)
    ct = lax.broadcasted_iota(jnp.int32, (e, e), 1)
    tril = (rt <= ct).astype(jnp.float32)
    pad_cum = lax.dot_general(padded, tril, (((1,), (0,)), ((), ())),
                              preferred_element_type=jnp.float32)
    pad_off = pad_cum - padded

    slot = cume + pad_off
    d1 = jnp.sum(jnp.where(mask1, slot, 0.0), axis=1).astype(jnp.int32)
    d2 = jnp.sum(jnp.where(mask2, slot, 0.0), axis=1).astype(jnp.int32)
    dest_ref[...] = jnp.concatenate(
        [d1.reshape(1, t), d2.reshape(1, t)], axis=0)

    wcat = jnp.concatenate([w1n, w2n], axis=0)
    wnb_ref[...] = wcat * jnp.ones((1, 128), jnp.float32)

    bi = lax.broadcasted_iota(jnp.int32, (_NB_PAD, e), 0).astype(jnp.float32)
    ge = (bi * _BLK >= pad_cum).astype(jnp.int32)
    eid = jnp.minimum(jnp.sum(ge, axis=1), e - 1)
    nbu = (lax.slice(pad_cum, (0, e - 1), (1, e)) * (1.0 / _BLK))
    nbu = nbu.astype(jnp.int32)[0, 0]
    pos = lax.broadcasted_iota(jnp.int32, (_NB_PAD,), 0)
    eid = jnp.where(pos == _NB_PAD - 1, nbu, eid)
    eid_ref[...] = eid.reshape(1, _NB_PAD)


def _moe_body(eid_ref, xs_ref, w1_ref, b1_ref, w2_ref, b2_ref, eo_ref,
              w1c_ref, w2c_ref, le_ref):
    bi = pl.program_id(0)
    nbu = eid_ref[_NB_PAD - 1]

    @pl.when(bi < nbu)
    def _compute():
        eid = eid_ref[bi]
        cvt = jnp.logical_or(bi == 0, eid != le_ref[0])
        xb = xs_ref[...].astype(jnp.bfloat16)
        dff = w1_ref.shape[2]
        fb = dff // 2
        acc = jnp.broadcast_to(b2_ref[0], eo_ref.shape).astype(jnp.float32)
        for fi in range(2):
            fs = pl.ds(fi * fb, fb)

            @pl.when(cvt)
            def _cvt():
                w1c_ref[:, fs] = w1_ref[0, :, fs].astype(jnp.bfloat16)
                w2c_ref[fs, :] = w2_ref[0, fs, :].astype(jnp.bfloat16)

            h = lax.dot_general(
                xb, w1c_ref[:, fs],
                (((1,), (0,)), ((), ())),
                preferred_element_type=jnp.float32)
            hg = jax.nn.gelu(h + b1_ref[0, :, fs])
            acc = acc + lax.dot_general(
                hg.astype(jnp.bfloat16), w2c_ref[fs, :],
                (((1,), (0,)), ((), ())),
                preferred_element_type=jnp.float32)
        eo_ref[...] = acc
        le_ref[0] = eid


def _make_dispatch(t, d, g, tpw, ch):
    mesh = plsc.VectorSubcoreMesh(core_axis_name="c", subcore_axis_name="s",
                                  num_cores=_NC, num_subcores=_NS)
    n = tpw // ch

    @functools.partial(
        pl.kernel, mesh=mesh,
        out_type=jax.ShapeDtypeStruct((g, d), jnp.float32),
        scratch_types=[[pltpu.VMEM((ch, d), jnp.float32)] * 2,
                       [pltpu.VMEM((ch,), jnp.int32)] * 2,
                       [pltpu.VMEM((ch,), jnp.int32)] * 2,
                       [pltpu.SemaphoreType.DMA] * 2,
                       [pltpu.SemaphoreType.DMA] * 2],
    )
    def dispatch(x_hbm, dest_hbm, xs_hbm, buf, idx1, idx2, gsem, ssem):
        wid = lax.axis_index("s") * _NC + lax.axis_index("c")
        tbase = wid * tpw

        def gather(ci):
            sb = ci % 2
            t0 = tbase + ci * ch
            pltpu.sync_copy(dest_hbm.at[pl.ds(t0, ch)], idx1[sb])
            pltpu.sync_copy(dest_hbm.at[pl.ds(t + t0, ch)], idx2[sb])
            return (pltpu.async_copy(x_hbm.at[pl.ds(t0, ch)], buf[sb],
                                     gsem[sb]),)

        def scatter(ci):
            sb = ci % 2
            return (pltpu.async_copy(buf[sb], xs_hbm.at[idx1[sb]], ssem[sb]),
                    pltpu.async_copy(buf[sb], xs_hbm.at[idx2[sb]], ssem[sb]))

        gps = {0: gather(0)}
        sps = {}
        for ci in range(n):
            if ci + 1 < n:
                if ci >= 1:
                    for cp in sps.pop(ci - 1):
                        cp.wait()
                gps[ci + 1] = gather(ci + 1)
            for cp in gps.pop(ci):
                cp.wait()
            sps[ci] = scatter(ci)
        for ci in sorted(sps):
            for cp in sps[ci]:
                cp.wait()

    return dispatch


def _make_combine(t, d, g, tpw, ch):
    mesh = plsc.VectorSubcoreMesh(core_axis_name="c", subcore_axis_name="s",
                                  num_cores=_NC, num_subcores=_NS)

    n = tpw // ch

    @functools.partial(
        pl.kernel, mesh=mesh,
        out_type=[jax.ShapeDtypeStruct((t, d), jnp.float32),
                  jax.ShapeDtypeStruct((t, d), jnp.float32)],
        scratch_types=[[pltpu.VMEM((ch, d), jnp.float32)] * 2,
                       [pltpu.VMEM((ch, d), jnp.float32)] * 2,
                       [pltpu.VMEM((ch,), jnp.int32)] * 2,
                       [pltpu.VMEM((ch,), jnp.int32)] * 2,
                       [pltpu.SemaphoreType.DMA] * 2,
                       [pltpu.SemaphoreType.DMA] * 2],
    )
    def combine(eo_hbm, dest_hbm, o1_hbm, o2_hbm, buf1, buf2, idx1, idx2,
                gsem, ssem):
        wid = lax.axis_index("s") * _NC + lax.axis_index("c")
        tbase = wid * tpw

        def gather(ci):
            sb = ci % 2
            t0 = tbase + ci * ch
            pltpu.sync_copy(dest_hbm.at[pl.ds(t0, ch)], idx1[sb])
            pltpu.sync_copy(dest_hbm.at[pl.ds(t + t0, ch)], idx2[sb])
            return (pltpu.async_copy(eo_hbm.at[idx1[sb]], buf1[sb], gsem[sb]),
                    pltpu.async_copy(eo_hbm.at[idx2[sb]], buf2[sb], gsem[sb]))

        def writeback(ci):
            sb = ci % 2
            t0 = tbase + ci * ch
            return (pltpu.async_copy(buf1[sb], o1_hbm.at[pl.ds(t0, ch)],
                                     ssem[sb]),
                    pltpu.async_copy(buf2[sb], o2_hbm.at[pl.ds(t0, ch)],
                                     ssem[sb]))

        gps = {0: gather(0)}
        sps = {}
        for ci in range(n):
            if ci + 1 < n:
                if ci >= 1:
                    for cp in sps.pop(ci - 1):
                        cp.wait()
                gps[ci + 1] = gather(ci + 1)
            for cp in gps.pop(ci):
                cp.wait()
            sps[ci] = writeback(ci)
        for ci in sorted(sps):
            for cp in sps[ci]:
                cp.wait()

    return combine


def _add_body(a_ref, b_ref, wa_ref, wb_ref, o_ref):
    o_ref[...] = (wa_ref[:, 0:1] * a_ref[...]
                  + wb_ref[:, 0:1] * b_ref[...])


def kernel(hidden_states, router_w1, router_b1, router_w2, router_b2,
           expert_w1, expert_b1, expert_w2, expert_b2):
    b, s, d = hidden_states.shape
    t = b * s
    e = router_w2.shape[1]
    dff = expert_w1.shape[2]
    rh = router_w1.shape[1]
    k = 2
    a = t * k
    g = a + e * _BLK
    nb = g // _BLK

    x = hidden_states.reshape(t, d)

    probs, dest, wnb, eid = pl.pallas_call(
        _router_body,
        grid=(1,),
        in_specs=[
            pl.BlockSpec((t, d), lambda i: (0, 0)),
            pl.BlockSpec((d, rh), lambda i: (0, 0)),
            pl.BlockSpec((1, rh), lambda i: (0, 0)),
            pl.BlockSpec((rh, e), lambda i: (0, 0)),
            pl.BlockSpec((1, e), lambda i: (0, 0)),
        ],
        out_specs=[
            pl.BlockSpec((t, e), lambda i: (0, 0)),
            pl.BlockSpec((k, t), lambda i: (0, 0)),
            pl.BlockSpec((a, 128), lambda i: (0, 0)),
            pl.BlockSpec((1, _NB_PAD), lambda i: (0, 0)),
        ],
        out_shape=[
            jax.ShapeDtypeStruct((t, e), jnp.float32),
            jax.ShapeDtypeStruct((k, t), jnp.int32),
            jax.ShapeDtypeStruct((a, 128), jnp.float32),
            jax.ShapeDtypeStruct((1, _NB_PAD), jnp.int32),
        ],
    )(x, router_w1, router_b1.reshape(1, rh), router_w2,
      router_b2.reshape(1, e))

    dest_flat = dest.reshape(a)
    eid_flat = eid.reshape(_NB_PAD)

    tpw = t // _NW
    xs = _make_dispatch(t, d, g, tpw, min(32, tpw))(x, dest_flat)

    grid_spec = pltpu.PrefetchScalarGridSpec(
        num_scalar_prefetch=1,
        grid=(nb,),
        scratch_shapes=[pltpu.VMEM((d, dff), jnp.bfloat16),
                        pltpu.VMEM((dff, d), jnp.bfloat16),
                        pltpu.SMEM((1,), jnp.int32)],
        in_specs=[
            pl.BlockSpec((_BLK, d), lambda bi, eid_r: (bi, 0)),
            pl.BlockSpec((1, d, dff), lambda bi, eid_r: (eid_r[bi], 0, 0)),
            pl.BlockSpec((1, 1, dff), lambda bi, eid_r: (eid_r[bi], 0, 0)),
            pl.BlockSpec((1, dff, d), lambda bi, eid_r: (eid_r[bi], 0, 0)),
            pl.BlockSpec((1, 1, d), lambda bi, eid_r: (eid_r[bi], 0, 0)),
        ],
        out_specs=pl.BlockSpec((_BLK, d), lambda bi, eid_r: (bi, 0)),
    )
    eo = pl.pallas_call(
        _moe_body,
        grid_spec=grid_spec,
        out_shape=jax.ShapeDtypeStruct((g, d), jnp.float32),
    )(eid_flat, xs, expert_w1, expert_b1.reshape(e, 1, dff), expert_w2,
      expert_b2.reshape(e, 1, d))

    eo1, eo2 = _make_combine(t, d, g, tpw, min(16, tpw))(eo, dest_flat)

    tb = min(1024, t)
    nblk_w = t // tb
    out = pl.pallas_call(
        _add_body,
        grid=(t // tb,),
        in_specs=[
            pl.BlockSpec((tb, d), lambda i: (i, 0)),
            pl.BlockSpec((tb, d), lambda i: (i, 0)),
            pl.BlockSpec((tb, 128), lambda i: (i, 0)),
            pl.BlockSpec((tb, 128), lambda i, _n=nblk_w: (i + _n, 0)),
        ],
        out_specs=pl.BlockSpec((tb, d), lambda i: (i, 0)),
        out_shape=jax.ShapeDtypeStruct((t, d), jnp.float32),
    )(eo1, eo2, wnb, wnb)

    return out.reshape(b, s, d), probs.reshape(b, s, e)


# speedup vs baseline: 1.2324x; 1.0127x over previous
import functools

import jax
import jax.numpy as jnp
from jax import lax
from jax.experimental import pallas as pl
from jax.experimental.pallas import tpu as pltpu
from jax.experimental.pallas import tpu_sc as plsc

_NC = 2
_NS = 16
_NW = _NC * _NS
_BLK = 512
_NB_PAD = 128


def _router_body(x_ref, w1_ref, b1_ref, w2_ref, b2_ref,
                 probs_ref, dest_ref, wnb_ref, eid_ref):
    x = x_ref[...]
    h = jnp.tanh(
        lax.dot_general(x, w1_ref[...], (((1,), (0,)), ((), ())),
                        preferred_element_type=jnp.float32) + b1_ref[...])
    logits = (
        lax.dot_general(h, w2_ref[...], (((1,), (0,)), ((), ())),
                        preferred_element_type=jnp.float32) + b2_ref[...])
    m = jnp.max(logits, axis=-1, keepdims=True)
    ex = jnp.exp(logits - m)
    probs = ex / jnp.sum(ex, axis=-1, keepdims=True)
    probs_ref[...] = probs

    t, e = probs.shape
    col = lax.broadcasted_iota(jnp.int32, (t, e), 1)
    m1 = jnp.max(probs, axis=-1, keepdims=True)
    i1 = jnp.argmax(probs, axis=-1)[:, None]
    probs_m = jnp.where(col == i1, -jnp.inf, probs)
    m2 = jnp.max(probs_m, axis=-1, keepdims=True)
    i2 = jnp.argmax(probs_m, axis=-1)[:, None]
    s = m1 + m2 + 1e-8
    w1n = m1 / s
    w2n = m2 / s

    mask1 = col == i1
    mask2 = col == i2
    mf = mask1.astype(jnp.float32) + mask2.astype(jnp.float32)
    cum = mf
    sh = 1
    while sh < t:
        shifted = jnp.concatenate(
            [jnp.zeros((sh, e), jnp.float32), lax.slice(cum, (0, 0), (t - sh, e))],
            axis=0)
        cum = cum + shifted
        sh *= 2
    cume = cum - mf
    counts = lax.slice(cum, (t - 1, 0), (t, e))
    padded = jnp.floor((counts + (_BLK - 1)) * (1.0 / _BLK)) * _BLK
    rt = lax.broadcasted_iota(jnp.int32, (e, e), 0)
    ct = lax.broadcasted_iota(jnp.int32, (e, e), 1)
    tril = (rt <= ct).astype(jnp.float32)
    pad_cum = lax.dot_general(padded, tril, (((1,), (0,)), ((), ())),
                              preferred_element_type=jnp.float32)
    pad_off = pad_cum - padded

    slot = cume + pad_off
    d1 = jnp.sum(jnp.where(mask1, slot, 0.0), axis=1).astype(jnp.int32)
    d2 = jnp.sum(jnp.where(mask2, slot, 0.0), axis=1).astype(jnp.int32)
    dest_ref[...] = jnp.concatenate(
        [d1.reshape(1, t), d2.reshape(1, t)], axis=0)

    wcat = jnp.concatenate([w1n, w2n], axis=0)
    wnb_ref[...] = wcat * jnp.ones((1, 128), jnp.float32)

    bi = lax.broadcasted_iota(jnp.int32, (_NB_PAD, e), 0).astype(jnp.float32)
    ge = (bi * _BLK >= pad_cum).astype(jnp.int32)
    eid = jnp.minimum(jnp.sum(ge, axis=1), e - 1)
    nbu = (lax.slice(pad_cum, (0, e - 1), (1, e)) * (1.0 / _BLK))
    nbu = nbu.astype(jnp.int32)[0, 0]
    pos = lax.broadcasted_iota(jnp.int32, (_NB_PAD,), 0)
    eid = jnp.where(pos == _NB_PAD - 1, nbu, eid)
    eid_ref[...] = eid.reshape(1, _NB_PAD)


def _moe_body(eid_ref, xs_ref, w1_ref, b1_ref, w2_ref, b2_ref, eo_ref,
              w1c_ref, w2c_ref, le_ref):
    bi = pl.program_id(0)
    nbu = eid_ref[_NB_PAD - 1]

    @pl.when(bi < nbu)
    def _compute():
        eid = eid_ref[bi]
        cvt = jnp.logical_or(bi == 0, eid != le_ref[0])
        xb = xs_ref[...].astype(jnp.bfloat16)
        dff = w1_ref.shape[2]
        fb = dff
        acc = jnp.broadcast_to(b2_ref[0], eo_ref.shape).astype(jnp.float32)
        for fi in range(1):
            fs = pl.ds(fi * fb, fb)

            @pl.when(cvt)
            def _cvt():
                w1c_ref[:, fs] = w1_ref[0, :, fs].astype(jnp.bfloat16)
                w2c_ref[fs, :] = w2_ref[0, fs, :].astype(jnp.bfloat16)

            h = lax.dot_general(
                xb, w1c_ref[:, fs],
                (((1,), (0,)), ((), ())),
                preferred_element_type=jnp.float32)
            hg = jax.nn.gelu(h + b1_ref[0, :, fs])
            acc = acc + lax.dot_general(
                hg.astype(jnp.bfloat16), w2c_ref[fs, :],
                (((1,), (0,)), ((), ())),
                preferred_element_type=jnp.float32)
        eo_ref[...] = acc
        le_ref[0] = eid


def _make_dispatch(t, d, g, tpw, ch):
    mesh = plsc.VectorSubcoreMesh(core_axis_name="c", subcore_axis_name="s",
                                  num_cores=_NC, num_subcores=_NS)
    n = tpw // ch

    @functools.partial(
        pl.kernel, mesh=mesh,
        out_type=jax.ShapeDtypeStruct((g, d), jnp.float32),
        scratch_types=[[pltpu.VMEM((ch, d), jnp.float32)] * 2,
                       [pltpu.VMEM((ch,), jnp.int32)] * 2,
                       [pltpu.VMEM((ch,), jnp.int32)] * 2,
                       [pltpu.SemaphoreType.DMA] * 2,
                       [pltpu.SemaphoreType.DMA] * 2],
    )
    def dispatch(x_hbm, dest_hbm, xs_hbm, buf, idx1, idx2, gsem, ssem):
        wid = lax.axis_index("s") * _NC + lax.axis_index("c")
        tbase = wid * tpw

        def gather(ci):
            sb = ci % 2
            t0 = tbase + ci * ch
            pltpu.sync_copy(dest_hbm.at[pl.ds(t0, ch)], idx1[sb])
            pltpu.sync_copy(dest_hbm.at[pl.ds(t + t0, ch)], idx2[sb])
            return (pltpu.async_copy(x_hbm.at[pl.ds(t0, ch)], buf[sb],
                                     gsem[sb]),)

        def scatter(ci):
            sb = ci % 2
            return (pltpu.async_copy(buf[sb], xs_hbm.at[idx1[sb]], ssem[sb]),
                    pltpu.async_copy(buf[sb], xs_hbm.at[idx2[sb]], ssem[sb]))

        gps = {0: gather(0)}
        sps = {}
        for ci in range(n):
            if ci + 1 < n:
                if ci >= 1:
                    for cp in sps.pop(ci - 1):
                        cp.wait()
                gps[ci + 1] = gather(ci + 1)
            for cp in gps.pop(ci):
                cp.wait()
            sps[ci] = scatter(ci)
        for ci in sorted(sps):
            for cp in sps[ci]:
                cp.wait()

    return dispatch


def _make_combine(t, d, g, tpw, ch):
    mesh = plsc.VectorSubcoreMesh(core_axis_name="c", subcore_axis_name="s",
                                  num_cores=_NC, num_subcores=_NS)

    n = tpw // ch

    @functools.partial(
        pl.kernel, mesh=mesh,
        out_type=[jax.ShapeDtypeStruct((t, d), jnp.float32),
                  jax.ShapeDtypeStruct((t, d), jnp.float32)],
        scratch_types=[[pltpu.VMEM((ch, d), jnp.float32)] * 2,
                       [pltpu.VMEM((ch, d), jnp.float32)] * 2,
                       [pltpu.VMEM((ch,), jnp.int32)] * 2,
                       [pltpu.VMEM((ch,), jnp.int32)] * 2,
                       [pltpu.SemaphoreType.DMA] * 2,
                       [pltpu.SemaphoreType.DMA] * 2],
    )
    def combine(eo_hbm, dest_hbm, o1_hbm, o2_hbm, buf1, buf2, idx1, idx2,
                gsem, ssem):
        wid = lax.axis_index("s") * _NC + lax.axis_index("c")
        tbase = wid * tpw

        def gather(ci):
            sb = ci % 2
            t0 = tbase + ci * ch
            pltpu.sync_copy(dest_hbm.at[pl.ds(t0, ch)], idx1[sb])
            pltpu.sync_copy(dest_hbm.at[pl.ds(t + t0, ch)], idx2[sb])
            return (pltpu.async_copy(eo_hbm.at[idx1[sb]], buf1[sb], gsem[sb]),
                    pltpu.async_copy(eo_hbm.at[idx2[sb]], buf2[sb], gsem[sb]))

        def writeback(ci):
            sb = ci % 2
            t0 = tbase + ci * ch
            return (pltpu.async_copy(buf1[sb], o1_hbm.at[pl.ds(t0, ch)],
                                     ssem[sb]),
                    pltpu.async_copy(buf2[sb], o2_hbm.at[pl.ds(t0, ch)],
                                     ssem[sb]))

        gps = {0: gather(0)}
        sps = {}
        for ci in range(n):
            if ci + 1 < n:
                if ci >= 1:
                    for cp in sps.pop(ci - 1):
                        cp.wait()
                gps[ci + 1] = gather(ci + 1)
            for cp in gps.pop(ci):
                cp.wait()
            sps[ci] = writeback(ci)
        for ci in sorted(sps):
            for cp in sps[ci]:
                cp.wait()

    return combine


def _add_body(a_ref, b_ref, wa_ref, wb_ref, o_ref):
    o_ref[...] = (wa_ref[:, 0:1] * a_ref[...]
                  + wb_ref[:, 0:1] * b_ref[...])


def kernel(hidden_states, router_w1, router_b1, router_w2, router_b2,
           expert_w1, expert_b1, expert_w2, expert_b2):
    b, s, d = hidden_states.shape
    t = b * s
    e = router_w2.shape[1]
    dff = expert_w1.shape[2]
    rh = router_w1.shape[1]
    k = 2
    a = t * k
    g = a + e * _BLK
    nb = g // _BLK

    x = hidden_states.reshape(t, d)

    probs, dest, wnb, eid = pl.pallas_call(
        _router_body,
        grid=(1,),
        in_specs=[
            pl.BlockSpec((t, d), lambda i: (0, 0)),
            pl.BlockSpec((d, rh), lambda i: (0, 0)),
            pl.BlockSpec((1, rh), lambda i: (0, 0)),
            pl.BlockSpec((rh, e), lambda i: (0, 0)),
            pl.BlockSpec((1, e), lambda i: (0, 0)),
        ],
        out_specs=[
            pl.BlockSpec((t, e), lambda i: (0, 0)),
            pl.BlockSpec((k, t), lambda i: (0, 0)),
            pl.BlockSpec((a, 128), lambda i: (0, 0)),
            pl.BlockSpec((1, _NB_PAD), lambda i: (0, 0)),
        ],
        out_shape=[
            jax.ShapeDtypeStruct((t, e), jnp.float32),
            jax.ShapeDtypeStruct((k, t), jnp.int32),
            jax.ShapeDtypeStruct((a, 128), jnp.float32),
            jax.ShapeDtypeStruct((1, _NB_PAD), jnp.int32),
        ],
    )(x, router_w1, router_b1.reshape(1, rh), router_w2,
      router_b2.reshape(1, e))

    dest_flat = dest.reshape(a)
    eid_flat = eid.reshape(_NB_PAD)

    tpw = t // _NW
    xs = _make_dispatch(t, d, g, tpw, min(32, tpw))(x, dest_flat)

    grid_spec = pltpu.PrefetchScalarGridSpec(
        num_scalar_prefetch=1,
        grid=(nb,),
        scratch_shapes=[pltpu.VMEM((d, dff), jnp.bfloat16),
                        pltpu.VMEM((dff, d), jnp.bfloat16),
                        pltpu.SMEM((1,), jnp.int32)],
        in_specs=[
            pl.BlockSpec((_BLK, d), lambda bi, eid_r: (bi, 0)),
            pl.BlockSpec((1, d, dff), lambda bi, eid_r: (eid_r[bi], 0, 0)),
            pl.BlockSpec((1, 1, dff), lambda bi, eid_r: (eid_r[bi], 0, 0)),
            pl.BlockSpec((1, dff, d), lambda bi, eid_r: (eid_r[bi], 0, 0)),
            pl.BlockSpec((1, 1, d), lambda bi, eid_r: (eid_r[bi], 0, 0)),
        ],
        out_specs=pl.BlockSpec((_BLK, d), lambda bi, eid_r: (bi, 0)),
    )
    eo = pl.pallas_call(
        _moe_body,
        grid_spec=grid_spec,
        out_shape=jax.ShapeDtypeStruct((g, d), jnp.float32),
    )(eid_flat, xs, expert_w1, expert_b1.reshape(e, 1, dff), expert_w2,
      expert_b2.reshape(e, 1, d))

    eo1, eo2 = _make_combine(t, d, g, tpw, min(16, tpw))(eo, dest_flat)

    tb = min(1024, t)
    nblk_w = t // tb
    out = pl.pallas_call(
        _add_body,
        grid=(t // tb,),
        in_specs=[
            pl.BlockSpec((tb, d), lambda i: (i, 0)),
            pl.BlockSpec((tb, d), lambda i: (i, 0)),
            pl.BlockSpec((tb, 128), lambda i: (i, 0)),
            pl.BlockSpec((tb, 128), lambda i, _n=nblk_w: (i + _n, 0)),
        ],
        out_specs=pl.BlockSpec((tb, d), lambda i: (i, 0)),
        out_shape=jax.ShapeDtypeStruct((t, d), jnp.float32),
    )(eo1, eo2, wnb, wnb)

    return out.reshape(b, s, d), probs.reshape(b, s, e)
